# Initial kernel scaffold; baseline (speedup 1.0000x reference)
import functools

import jax
import jax.numpy as jnp
from jax import lax
from jax.experimental import pallas as pl
from jax.experimental.pallas import tpu as pltpu
from jax.experimental.pallas import tpu_sc as plsc

RAD_D = 60
OUT_D = 64
LANES = 16
NUM_WORKERS = 32
CHUNK = 512
GATHER_SLICE = 128

_MAGIC = jnp.int32(0x5F3759DF)


def _rsqrt16(x):
    y = plsc.bitcast(_MAGIC - (plsc.bitcast(x, jnp.int32) >> 1), jnp.float32)
    for _ in range(3):
        y = y * (1.5 - 0.5 * x * y * y)
    return y


def _make_sc_call(n_rows):
    rows_per_w = n_rows // NUM_WORKERS
    n_chunks = rows_per_w // CHUNK
    assert rows_per_w % CHUNK == 0 and CHUNK % GATHER_SLICE == 0

    mesh = plsc.VectorSubcoreMesh(core_axis_name="c", subcore_axis_name="s")

    @functools.partial(
        pl.kernel,
        out_type=jax.ShapeDtypeStruct((n_rows, OUT_D), jnp.float32),
        mesh=mesh,
        scratch_types=[
            pltpu.VMEM((CHUNK,), jnp.int32),
            pltpu.VMEM((CHUNK, RAD_D), jnp.float32),
            pltpu.VMEM((CHUNK, OUT_D), jnp.float32),
            pltpu.VMEM((CHUNK * 4,), jnp.float32),
            pltpu.SemaphoreType.DMA,
        ],
    )
    def sc_call(idx_hbm, pos_hbm, table_hbm, out_hbm, idx_v, rows_v, out_v,
                pos_v, sem):
        wid = lax.axis_index("s") * 2 + lax.axis_index("c")
        base = wid * rows_per_w
        iota = lax.iota(jnp.int32, LANES)

        def chunk_body(ci, carry):
            row0 = pl.multiple_of(base + ci * CHUNK, CHUNK)
            pltpu.sync_copy(idx_hbm.at[pl.ds(row0, CHUNK)], idx_v)
            pltpu.sync_copy(pos_hbm.at[pl.ds(row0 * 4, CHUNK * 4)], pos_v)
            for g in range(CHUNK // GATHER_SLICE):
                pltpu.async_copy(
                    table_hbm.at[idx_v.at[pl.ds(g * GATHER_SLICE,
                                                GATHER_SLICE)]],
                    rows_v.at[pl.ds(g * GATHER_SLICE, GATHER_SLICE), :],
                    sem,
                ).wait()

            def group_body(gi, carry2):
                r0 = gi * LANES
                rowids = r0 + iota
                acc = jnp.zeros((LANES,), jnp.float32)
                for j in range(RAD_D):
                    col = jnp.full((LANES,), j, jnp.int32)
                    v = plsc.load_gather(rows_v, [rowids, col])
                    acc = acc + v * v
                scale = jnp.minimum(
                    jnp.float32(1.0), _rsqrt16(jnp.maximum(acc, 1e-30)))
                for j in range(RAD_D):
                    col = jnp.full((LANES,), j, jnp.int32)
                    v = plsc.load_gather(rows_v, [rowids, col])
                    plsc.store_scatter(out_v, [rowids, col], v * scale)
                for t in range(4):
                    w = r0 * 4 + t * LANES + iota
                    pv = plsc.load_gather(pos_v, [w])
                    plsc.store_scatter(out_v, [w >> 2, 60 + (w & 3)], pv)
                return carry2

            lax.fori_loop(0, CHUNK // LANES, group_body, 0)
            pltpu.sync_copy(out_v, out_hbm.at[pl.ds(row0, CHUNK), :])
            return carry

        lax.fori_loop(0, n_chunks, chunk_body, 0)

    return sc_call


def kernel(indices, positions, table):
    b, l = indices.shape
    n_rows = b * l
    out = _make_sc_call(n_rows)(
        indices.reshape(n_rows),
        positions.reshape(n_rows * 4),
        table,
    )
    return out.reshape(b, l, OUT_D)


# speedup vs baseline: 1.0722x; 1.0722x over previous
import functools

import jax
import jax.numpy as jnp
from jax import lax
from jax.experimental import pallas as pl
from jax.experimental.pallas import tpu as pltpu
from jax.experimental.pallas import tpu_sc as plsc

RAD_D = 60
OUT_D = 64
LANES = 16
NUM_WORKERS = 32
CHUNK = 512
GATHER_SLICE = 128

def _rsqrt16(x):
    y = plsc.bitcast(0x5F3759DF - (plsc.bitcast(x, jnp.int32) >> 1),
                     jnp.float32)
    for _ in range(3):
        y = y * (1.5 - 0.5 * x * y * y)
    return y


def _make_sc_call(n_rows):
    rows_per_w = n_rows // NUM_WORKERS
    n_chunks = rows_per_w // CHUNK
    assert rows_per_w % CHUNK == 0 and CHUNK % GATHER_SLICE == 0

    mesh = plsc.VectorSubcoreMesh(core_axis_name="c", subcore_axis_name="s")

    @functools.partial(
        pl.kernel,
        out_type=jax.ShapeDtypeStruct((n_rows, OUT_D), jnp.float32),
        mesh=mesh,
        compiler_params=pltpu.CompilerParams(
            needs_layout_passes=False, use_tc_tiling_on_sc=False),
        scratch_types=[
            pltpu.VMEM((CHUNK // GATHER_SLICE, GATHER_SLICE), jnp.int32),
            pltpu.VMEM((CHUNK, OUT_D), jnp.float32),
            pltpu.VMEM((CHUNK * 4,), jnp.float32),
            pltpu.SemaphoreType.DMA,
        ],
    )
    def sc_call(idx_hbm, pos_hbm, table_hbm, out_hbm, idx_v, out_v,
                pos_v, sem):
        wid = lax.axis_index("s") * 2 + lax.axis_index("c")
        base = wid * rows_per_w
        iota = lax.iota(jnp.int32, LANES)

        def chunk_body(ci, carry):
            row0 = pl.multiple_of(base + ci * CHUNK, CHUNK)
            blk0 = pl.multiple_of(row0 // GATHER_SLICE,
                                  CHUNK // GATHER_SLICE)
            pltpu.sync_copy(
                idx_hbm.at[pl.ds(blk0, CHUNK // GATHER_SLICE), :], idx_v)
            pltpu.sync_copy(pos_hbm.at[pl.ds(row0 * 4, CHUNK * 4)], pos_v)
            for g in range(CHUNK // GATHER_SLICE):
                pltpu.async_copy(
                    table_hbm.at[idx_v.at[g]],
                    out_v.at[pl.ds(g * GATHER_SLICE, GATHER_SLICE), :],
                    sem,
                ).wait()

            def group_body(gi, carry2):
                r0 = gi * LANES
                rowids = r0 + iota
                acc = jnp.zeros((LANES,), jnp.float32)
                for j in range(RAD_D):
                    col = jnp.full((LANES,), j, jnp.int32)
                    v = plsc.load_gather(out_v, [rowids, col])
                    acc = acc + v * v
                scale = jnp.minimum(
                    jnp.float32(1.0), _rsqrt16(jnp.maximum(acc, 1e-30)))
                for j in range(RAD_D):
                    col = jnp.full((LANES,), j, jnp.int32)
                    v = plsc.load_gather(out_v, [rowids, col])
                    plsc.store_scatter(out_v, [rowids, col], v * scale)
                for t in range(4):
                    w = r0 * 4 + t * LANES + iota
                    pv = plsc.load_gather(pos_v, [w])
                    plsc.store_scatter(out_v, [w >> 2, 60 + (w & 3)], pv)
                return carry2

            lax.fori_loop(0, CHUNK // LANES, group_body, 0)
            pltpu.sync_copy(out_v, out_hbm.at[pl.ds(row0, CHUNK), :])
            return carry

        lax.fori_loop(0, n_chunks, chunk_body, 0)

    return sc_call


def kernel(indices, positions, table):
    b, l = indices.shape
    n_rows = b * l
    table64 = jnp.concatenate(
        [table, jnp.zeros((table.shape[0], OUT_D - RAD_D), table.dtype)],
        axis=1)
    out = _make_sc_call(n_rows)(
        indices.reshape(n_rows // GATHER_SLICE, GATHER_SLICE),
        positions.reshape(n_rows * 4),
        table64,
    )
    return out.reshape(b, l, OUT_D)


# speedup vs baseline: 1.1545x; 1.0767x over previous
import functools

import jax
import jax.numpy as jnp
from jax import lax
from jax.experimental import pallas as pl
from jax.experimental.pallas import tpu as pltpu
from jax.experimental.pallas import tpu_sc as plsc

RAD_D = 60
OUT_D = 64
LANES = 16
NUM_WORKERS = 32
CHUNK = 256
GATHER_SLICE = 128
NBUF = 4
LOOKAHEAD = 2


def _rsqrt16(x):
    y = plsc.bitcast(0x5F3759DF - (plsc.bitcast(x, jnp.int32) >> 1),
                     jnp.float32)
    for _ in range(3):
        y = y * (1.5 - 0.5 * x * y * y)
    return y


def _make_sc_call(n_rows):
    rows_per_w = n_rows // NUM_WORKERS
    n_chunks = rows_per_w // CHUNK
    blk_per_chunk = CHUNK // GATHER_SLICE
    blk_per_w = rows_per_w // GATHER_SLICE
    assert rows_per_w % CHUNK == 0 and CHUNK % GATHER_SLICE == 0
    assert n_chunks % NBUF == 0 and NBUF > LOOKAHEAD

    mesh = plsc.VectorSubcoreMesh(core_axis_name="c", subcore_axis_name="s")

    @functools.partial(
        pl.kernel,
        out_type=jax.ShapeDtypeStruct((n_rows, OUT_D), jnp.float32),
        mesh=mesh,
        compiler_params=pltpu.CompilerParams(
            needs_layout_passes=False, use_tc_tiling_on_sc=False),
        scratch_types=[
            pltpu.VMEM((blk_per_w, GATHER_SLICE), jnp.int32),
            [pltpu.VMEM((CHUNK, OUT_D), jnp.float32)] * NBUF,
            [pltpu.VMEM((CHUNK * 4,), jnp.float32)] * NBUF,
            [pltpu.SemaphoreType.DMA] * NBUF,
            [pltpu.SemaphoreType.DMA] * NBUF,
            [pltpu.SemaphoreType.DMA] * NBUF,
        ],
    )
    def sc_call(idx_hbm, pos_hbm, table_hbm, out_hbm, idx_v, tiles, poss,
                gsems, psems, osems):
        wid = lax.axis_index("s") * 2 + lax.axis_index("c")
        base = wid * rows_per_w
        iota = lax.iota(jnp.int32, LANES)

        pltpu.sync_copy(
            idx_hbm.at[pl.ds(pl.multiple_of(wid * blk_per_w, blk_per_w),
                             blk_per_w), :],
            idx_v)

        def fire(ci, b):
            row0 = pl.multiple_of(base + ci * CHUNK, CHUNK)
            pltpu.async_copy(
                pos_hbm.at[pl.ds(row0 * 4, CHUNK * 4)], poss[b], psems[b])
            for g in range(blk_per_chunk):
                pltpu.async_copy(
                    table_hbm.at[idx_v.at[ci * blk_per_chunk + g]],
                    tiles[b].at[pl.ds(g * GATHER_SLICE, GATHER_SLICE), :],
                    gsems[b],
                )

        def wait_in(ci, b):
            pltpu.make_async_copy(
                pos_hbm.at[pl.ds(0, CHUNK * 4)], poss[b], psems[b]).wait()
            for g in range(blk_per_chunk):
                pltpu.make_async_copy(
                    table_hbm.at[idx_v.at[g]],
                    tiles[b].at[pl.ds(g * GATHER_SLICE, GATHER_SLICE), :],
                    gsems[b],
                ).wait()

        def wait_out(b):
            pltpu.make_async_copy(
                tiles[b], out_hbm.at[pl.ds(0, CHUNK), :], osems[b]).wait()

        for p in range(LOOKAHEAD):
            fire(p, p)

        def quad_body(qi, carry):
            for b in range(NBUF):
                ci = qi * NBUF + b
                wait_in(ci, b)
                tile = tiles[b]
                pos_v = poss[b]

                def group_body(gi, carry2):
                    r0 = gi * LANES
                    rowids = r0 + iota
                    acc = jnp.zeros((LANES,), jnp.float32)
                    for j in range(RAD_D):
                        col = jnp.full((LANES,), j, jnp.int32)
                        v = plsc.load_gather(tile, [rowids, col])
                        acc = acc + v * v
                    scale = jnp.minimum(
                        jnp.float32(1.0), _rsqrt16(jnp.maximum(acc, 1e-30)))
                    for j in range(RAD_D):
                        col = jnp.full((LANES,), j, jnp.int32)
                        v = plsc.load_gather(tile, [rowids, col])
                        plsc.store_scatter(tile, [rowids, col], v * scale)
                    for t in range(4):
                        w = r0 * 4 + t * LANES + iota
                        pv = plsc.load_gather(pos_v, [w])
                        plsc.store_scatter(tile, [w >> 2, 60 + (w & 3)], pv)
                    return carry2

                lax.fori_loop(0, CHUNK // LANES, group_body, 0)

                row0 = pl.multiple_of(base + ci * CHUNK, CHUNK)
                pltpu.async_copy(
                    tile, out_hbm.at[pl.ds(row0, CHUNK), :], osems[b])

                nb = (b + LOOKAHEAD) % NBUF
                nci = ci + LOOKAHEAD

                @pl.when(nci < n_chunks)
                def _():
                    @pl.when(nci >= NBUF)
                    def _():
                        wait_out(nb)
                    fire(nci, nb)
            return carry

        lax.fori_loop(0, n_chunks // NBUF, quad_body, 0)
        for b in range(NBUF):
            wait_out(b)

    return sc_call


def kernel(indices, positions, table):
    b, l = indices.shape
    n_rows = b * l
    table64 = jnp.concatenate(
        [table, jnp.zeros((table.shape[0], OUT_D - RAD_D), table.dtype)],
        axis=1)
    out = _make_sc_call(n_rows)(
        indices.reshape(n_rows // GATHER_SLICE, GATHER_SLICE),
        positions.reshape(n_rows * 4),
        table64,
    )
    return out.reshape(b, l, OUT_D)


# speedup vs baseline: 2.1866x; 1.8941x over previous
import functools

import jax
import jax.numpy as jnp
from jax import lax
from jax.experimental import pallas as pl
from jax.experimental.pallas import tpu as pltpu
from jax.experimental.pallas import tpu_sc as plsc

RAD_D = 60
OUT_D = 64
LANES = 16
NUM_WORKERS = 32
CHUNK = 256
GATHER_SLICE = 128
NBUF = 4
LOOKAHEAD = 2


def _rsqrt16(x):
    y = plsc.bitcast(0x5F3759DF - (plsc.bitcast(x, jnp.int32) >> 1),
                     jnp.float32)
    for _ in range(3):
        y = y * (1.5 - 0.5 * x * y * y)
    return y


def _make_sc_call(n_rows):
    rows_per_w = n_rows // NUM_WORKERS
    n_chunks = rows_per_w // CHUNK
    blk_per_chunk = CHUNK // GATHER_SLICE
    blk_per_w = rows_per_w // GATHER_SLICE
    assert rows_per_w % CHUNK == 0 and CHUNK % GATHER_SLICE == 0
    assert n_chunks % NBUF == 0 and NBUF > LOOKAHEAD

    mesh = plsc.VectorSubcoreMesh(core_axis_name="c", subcore_axis_name="s")

    @functools.partial(
        pl.kernel,
        out_type=jax.ShapeDtypeStruct((n_rows, OUT_D), jnp.float32),
        mesh=mesh,
        compiler_params=pltpu.CompilerParams(
            needs_layout_passes=False, use_tc_tiling_on_sc=False),
        scratch_types=[
            pltpu.VMEM((blk_per_w, GATHER_SLICE), jnp.int32),
            [pltpu.VMEM((CHUNK, OUT_D), jnp.float32)] * NBUF,
            [pltpu.VMEM((CHUNK * 4,), jnp.float32)] * NBUF,
            [pltpu.SemaphoreType.DMA] * NBUF,
            [pltpu.SemaphoreType.DMA] * NBUF,
            [pltpu.SemaphoreType.DMA] * NBUF,
        ],
    )
    def sc_call(idx_hbm, pos_hbm, table_hbm, out_hbm, idx_v, tiles, poss,
                gsems, psems, osems):
        wid = lax.axis_index("s") * 2 + lax.axis_index("c")
        base = wid * rows_per_w
        iota = lax.iota(jnp.int32, LANES)

        pltpu.sync_copy(
            idx_hbm.at[pl.ds(pl.multiple_of(wid * blk_per_w, blk_per_w),
                             blk_per_w), :],
            idx_v)

        def fire(ci, b):
            row0 = pl.multiple_of(base + ci * CHUNK, CHUNK)
            pltpu.async_copy(
                pos_hbm.at[pl.ds(row0 * 4, CHUNK * 4)], poss[b], psems[b])
            for g in range(blk_per_chunk):
                pltpu.async_copy(
                    table_hbm.at[idx_v.at[ci * blk_per_chunk + g]],
                    tiles[b].at[pl.ds(g * GATHER_SLICE, GATHER_SLICE), :],
                    gsems[b],
                )

        def wait_in(ci, b):
            pltpu.make_async_copy(
                pos_hbm.at[pl.ds(0, CHUNK * 4)], poss[b], psems[b]).wait()
            for g in range(blk_per_chunk):
                pltpu.make_async_copy(
                    table_hbm.at[idx_v.at[g]],
                    tiles[b].at[pl.ds(g * GATHER_SLICE, GATHER_SLICE), :],
                    gsems[b],
                ).wait()

        def wait_out(b):
            pltpu.make_async_copy(
                tiles[b], out_hbm.at[pl.ds(0, CHUNK), :], osems[b]).wait()

        for p in range(LOOKAHEAD):
            fire(p, p)

        def quad_body(qi, carry):
            for b in range(NBUF):
                ci = qi * NBUF + b
                wait_in(ci, b)
                tile = tiles[b]
                pos_v = poss[b]

                def group_body(gi, carry2):
                    r0 = gi * LANES
                    rowids = r0 + iota
                    acc = jnp.zeros((LANES,), jnp.float32)
                    for j in range(OUT_D):
                        cj = (iota + j) & 63 if j > 63 - LANES else iota + j
                        v = plsc.load_gather(tile, [rowids, cj])
                        acc = acc + v * v
                    scale = jnp.minimum(
                        jnp.float32(1.0), _rsqrt16(jnp.maximum(acc, 1e-30)))
                    for j in range(OUT_D):
                        cj = (iota + j) & 63 if j > 63 - LANES else iota + j
                        v = plsc.load_gather(tile, [rowids, cj])
                        plsc.store_scatter(tile, [rowids, cj], v * scale)
                    for t in range(4):
                        w = r0 * 4 + t * LANES + iota
                        pv = plsc.load_gather(pos_v, [w])
                        plsc.store_scatter(tile, [w >> 2, 60 + (w & 3)], pv)
                    return carry2

                lax.fori_loop(0, CHUNK // LANES, group_body, 0)

                row0 = pl.multiple_of(base + ci * CHUNK, CHUNK)
                pltpu.async_copy(
                    tile, out_hbm.at[pl.ds(row0, CHUNK), :], osems[b])

                nb = (b + LOOKAHEAD) % NBUF
                nci = ci + LOOKAHEAD

                @pl.when(nci < n_chunks)
                def _():
                    @pl.when(nci >= NBUF)
                    def _():
                        wait_out(nb)
                    fire(nci, nb)
            return carry

        lax.fori_loop(0, n_chunks // NBUF, quad_body, 0)
        for b in range(NBUF):
            wait_out(b)

    return sc_call


def kernel(indices, positions, table):
    b, l = indices.shape
    n_rows = b * l
    table64 = jnp.concatenate(
        [table, jnp.zeros((table.shape[0], OUT_D - RAD_D), table.dtype)],
        axis=1)
    out = _make_sc_call(n_rows)(
        indices.reshape(n_rows // GATHER_SLICE, GATHER_SLICE),
        positions.reshape(n_rows * 4),
        table64,
    )
    return out.reshape(b, l, OUT_D)
